# Initial kernel scaffold; baseline (speedup 1.0000x reference)
import jax
import jax.numpy as jnp
from jax.experimental import pallas as pl


def kernel(x, edge_index, edge_type, idx, attt, Wrel1, Wroot1, b1, Wrel2, Wroot2, b2, Wg1, bg1, Wg2, bg2, M1, mb1, M2, mb2, M3, mb3, M4, mb4, features3):
    raise NotImplementedError("write your pallas kernel here")


# speedup vs baseline: 7.3534x; 7.3534x over previous
import functools

import jax
import jax.numpy as jnp
from jax import lax
from jax.experimental import pallas as pl
from jax.experimental.pallas import tpu as pltpu
from jax.experimental.pallas import tpu_sc as plsc

N = 10000
E = 320000
B = 100000

NC, NS = 2, 16
NW = NC * NS
CH = 128

E_PAD = ((E + NW * CH - 1) // (NW * CH)) * (NW * CH)
B_PAD = ((B + NW * CH - 1) // (NW * CH)) * (NW * CH)
AR = N + 48
A2 = N + 112

BN = 1000
GRID_N = N // BN


def _make_sc_segsum(table_rows, width, acc_rows, e_pad):
    epw = e_pad // NW
    n_ch = epw // CH
    stripe = acc_rows // NS
    e_half = e_pad // NC
    mesh = plsc.VectorSubcoreMesh(core_axis_name="c", subcore_axis_name="s")

    @functools.partial(
        pl.kernel, mesh=mesh,
        compiler_params=pltpu.CompilerParams(use_tc_tiling_on_sc=False),
        out_type=jax.ShapeDtypeStruct((NC, acc_rows, width), jnp.float32),
        scratch_types=[
            pltpu.VMEM((CH,), jnp.int32),
            pltpu.VMEM((CH,), jnp.int32),
            pltpu.VMEM((CH, width), jnp.float32),
            pltpu.VMEM_SHARED((acc_rows, width), jnp.float32),
            pltpu.SemaphoreType.DMA,
        ],
    )
    def k(table, gidx, sidx, zeros, out, gv, sv, rows, acc, sem):
        c = lax.axis_index("c")
        s = lax.axis_index("s")
        pltpu.sync_copy(zeros, acc.at[pl.ds(s * stripe, stripe)])
        plsc.subcore_barrier()
        base = c * e_half + s * epw

        def body(j, carry):
            off = base + j * CH
            pltpu.sync_copy(gidx.at[pl.ds(off, CH)], gv)
            pltpu.sync_copy(sidx.at[pl.ds(off, CH)], sv)
            pltpu.async_copy(table.at[gv], rows, sem).wait()
            pltpu.sync_copy(rows, acc.at[sv], add=True)
            return carry

        lax.fori_loop(0, n_ch, body, 0)
        plsc.subcore_barrier()
        pltpu.sync_copy(acc.at[pl.ds(s * stripe, stripe)],
                        out.at[c, pl.ds(s * stripe, stripe)])

    return k


def _make_sc_pair_gather(width):
    ppw = B_PAD // NW
    n_ch = ppw // CH
    kk = width // 16
    mesh = plsc.VectorSubcoreMesh(core_axis_name="c", subcore_axis_name="s")

    @functools.partial(
        pl.kernel, mesh=mesh,
        compiler_params=pltpu.CompilerParams(use_tc_tiling_on_sc=False),
        out_type=jax.ShapeDtypeStruct((B_PAD, width), jnp.float32),
        scratch_types=[
            pltpu.VMEM((CH,), jnp.int32),
            pltpu.VMEM((CH,), jnp.int32),
            pltpu.VMEM((CH, width), jnp.float32),
            pltpu.VMEM((CH, width), jnp.float32),
            pltpu.SemaphoreType.DMA,
            pltpu.SemaphoreType.DMA,
        ],
    )
    def k(pa, pb, aidx, bidx, out, av, bv, ra, rb, sema, semb):
        c = lax.axis_index("c")
        s = lax.axis_index("s")
        base = (s * NC + c) * ppw

        def body(j, carry):
            off = base + j * CH
            pltpu.sync_copy(aidx.at[pl.ds(off, CH)], av)
            pltpu.sync_copy(bidx.at[pl.ds(off, CH)], bv)
            cpa = pltpu.async_copy(pa.at[av], ra, sema)
            cpb = pltpu.async_copy(pb.at[bv], rb, semb)
            cpa.wait()
            cpb.wait()

            def add_row(r, carry2):
                for q in range(kk):
                    ra[r, pl.ds(q * 16, 16)] = (
                        ra[r, pl.ds(q * 16, 16)] + rb[r, pl.ds(q * 16, 16)])
                return carry2

            lax.fori_loop(0, CH, add_row, 0)
            pltpu.sync_copy(ra, out.at[pl.ds(off, CH)])
            return carry

        lax.fori_loop(0, n_ch, body, 0)

    return k


_sc_seg1 = _make_sc_segsum(2 * N, 80, 2 * AR, E_PAD)
_sc_seg2 = _make_sc_segsum(N, 64, A2, E_PAD)
_sc_seg3 = _make_sc_segsum(2 * N, 32, 2 * AR, E_PAD)
_sc_seg4 = _make_sc_segsum(N, 32, A2, E_PAD)
_sc_pair = _make_sc_pair_gather(256)


def _k1_body(x_ref, w_ref, t_ref):
    xb = x_ref[...]
    one = jnp.ones((BN, 1), jnp.float32)
    zer = jnp.zeros((BN, 15), jnp.float32)
    t_ref[0] = jnp.concatenate(
        [jnp.dot(xb, w_ref[0], preferred_element_type=jnp.float32), one, zer], -1)
    t_ref[1] = jnp.concatenate(
        [jnp.dot(xb, w_ref[1], preferred_element_type=jnp.float32), one, zer], -1)


def _k1(x, Wrel1):
    return pl.pallas_call(
        _k1_body,
        grid=(GRID_N,),
        in_specs=[
            pl.BlockSpec((BN, 128), lambda g: (g, 0)),
            pl.BlockSpec((2, 128, 64), lambda g: (0, 0, 0)),
        ],
        out_specs=pl.BlockSpec((2, BN, 80), lambda g: (0, g, 0)),
        out_shape=jax.ShapeDtypeStruct((2, N, 80), jnp.float32),
    )(x, Wrel1)


def _k2_body(x_ref, s1_ref, wr_ref, b1_ref, wg_ref, w2_ref,
             x1_ref, hg1_ref, t2_ref, aux_ref):
    S0 = s1_ref[0, 0] + s1_ref[1, 0]
    S1 = s1_ref[0, 1] + s1_ref[1, 1]
    c0 = S0[:, 64]
    c1 = S1[:, 64]
    dinv = lax.rsqrt(c0 + c1 + 1.0)
    ic0 = 1.0 / jnp.maximum(c0, 1.0)
    ic1 = 1.0 / jnp.maximum(c1, 1.0)
    xb = x_ref[...]
    x1 = jnp.dot(xb, wr_ref[...], preferred_element_type=jnp.float32) + b1_ref[...]
    x1 = x1 + S0[:, :64] * ic0[:, None] + S1[:, :64] * ic1[:, None]
    x1 = jnp.maximum(x1, 0.0)
    x1_ref[...] = x1
    hg1_ref[...] = dinv[:, None] * jnp.dot(xb, wg_ref[...],
                                           preferred_element_type=jnp.float32)
    t2_ref[0] = jnp.dot(x1, w2_ref[0], preferred_element_type=jnp.float32)
    t2_ref[1] = jnp.dot(x1, w2_ref[1], preferred_element_type=jnp.float32)
    aux_ref[...] = jnp.concatenate(
        [dinv[:, None], ic0[:, None], ic1[:, None],
         jnp.zeros((BN, 5), jnp.float32)], -1)


def _k2(x, S1, Wroot1, b1, Wg1, Wrel2):
    return pl.pallas_call(
        _k2_body,
        grid=(GRID_N,),
        in_specs=[
            pl.BlockSpec((BN, 128), lambda g: (g, 0)),
            pl.BlockSpec((2, 2, BN, 80), lambda g: (0, 0, g, 0)),
            pl.BlockSpec((128, 64), lambda g: (0, 0)),
            pl.BlockSpec((1, 64), lambda g: (0, 0)),
            pl.BlockSpec((128, 64), lambda g: (0, 0)),
            pl.BlockSpec((2, 64, 32), lambda g: (0, 0, 0)),
        ],
        out_specs=[
            pl.BlockSpec((BN, 64), lambda g: (g, 0)),
            pl.BlockSpec((BN, 64), lambda g: (g, 0)),
            pl.BlockSpec((2, BN, 32), lambda g: (0, g, 0)),
            pl.BlockSpec((BN, 8), lambda g: (g, 0)),
        ],
        out_shape=[
            jax.ShapeDtypeStruct((N, 64), jnp.float32),
            jax.ShapeDtypeStruct((N, 64), jnp.float32),
            jax.ShapeDtypeStruct((2, N, 32), jnp.float32),
            jax.ShapeDtypeStruct((N, 8), jnp.float32),
        ],
    )(x, S1, Wroot1, b1, Wg1, Wrel2)


def _k3_body(g1_ref, hg1_ref, aux_ref, bg1_ref, wg2_ref, x11_ref, hg2_ref):
    dinv = aux_ref[...][:, 0]
    G = g1_ref[0] + g1_ref[1]
    x11 = jnp.maximum(dinv[:, None] * (G + hg1_ref[...]) + bg1_ref[...], 0.0)
    x11_ref[...] = x11
    hg2_ref[...] = dinv[:, None] * jnp.dot(x11, wg2_ref[...],
                                           preferred_element_type=jnp.float32)


def _k3(G1, hg1, aux, bg1, Wg2):
    return pl.pallas_call(
        _k3_body,
        grid=(GRID_N,),
        in_specs=[
            pl.BlockSpec((2, BN, 64), lambda g: (0, g, 0)),
            pl.BlockSpec((BN, 64), lambda g: (g, 0)),
            pl.BlockSpec((BN, 8), lambda g: (g, 0)),
            pl.BlockSpec((1, 64), lambda g: (0, 0)),
            pl.BlockSpec((64, 32), lambda g: (0, 0)),
        ],
        out_specs=[
            pl.BlockSpec((BN, 64), lambda g: (g, 0)),
            pl.BlockSpec((BN, 32), lambda g: (g, 0)),
        ],
        out_shape=[
            jax.ShapeDtypeStruct((N, 64), jnp.float32),
            jax.ShapeDtypeStruct((N, 32), jnp.float32),
        ],
    )(G1, hg1, aux, bg1, Wg2)


def _k4_body(s2_ref, x1_ref, wr2_ref, b2_ref, aux_ref, g2_ref, hg2_ref,
             bg2_ref, x11_ref, f3_ref, mab_ref, x2_ref, pa_ref, pb_ref):
    aux = aux_ref[...]
    dinv = aux[:, 0]
    ic0 = aux[:, 1]
    ic1 = aux[:, 2]
    S0 = s2_ref[0, 0] + s2_ref[1, 0]
    S1 = s2_ref[0, 1] + s2_ref[1, 1]
    x1 = x1_ref[...]
    x2 = jnp.dot(x1, wr2_ref[...], preferred_element_type=jnp.float32) + b2_ref[...]
    x2 = x2 + S0 * ic0[:, None] + S1 * ic1[:, None]
    x2_ref[...] = x2
    G2 = g2_ref[0] + g2_ref[1]
    x21 = dinv[:, None] * (G2 + hg2_ref[...]) + bg2_ref[...]
    Z = jnp.concatenate([x11_ref[...], x21, x1, x2, f3_ref[...]], -1)
    pa_ref[...] = jnp.dot(Z, mab_ref[0], preferred_element_type=jnp.float32)
    pb_ref[...] = jnp.dot(Z, mab_ref[1], preferred_element_type=jnp.float32)


def _k4(S2, x1_o, Wroot2, b2, aux, G2, hg2, bg2, x1_o1, f3, Mab):
    return pl.pallas_call(
        _k4_body,
        grid=(GRID_N,),
        in_specs=[
            pl.BlockSpec((2, 2, BN, 32), lambda g: (0, 0, g, 0)),
            pl.BlockSpec((BN, 64), lambda g: (g, 0)),
            pl.BlockSpec((64, 32), lambda g: (0, 0)),
            pl.BlockSpec((1, 32), lambda g: (0, 0)),
            pl.BlockSpec((BN, 8), lambda g: (g, 0)),
            pl.BlockSpec((2, BN, 32), lambda g: (0, g, 0)),
            pl.BlockSpec((BN, 32), lambda g: (g, 0)),
            pl.BlockSpec((1, 32), lambda g: (0, 0)),
            pl.BlockSpec((BN, 64), lambda g: (g, 0)),
            pl.BlockSpec((BN, 512), lambda g: (g, 0)),
            pl.BlockSpec((2, 704, 256), lambda g: (0, 0, 0)),
        ],
        out_specs=[
            pl.BlockSpec((BN, 32), lambda g: (g, 0)),
            pl.BlockSpec((BN, 256), lambda g: (g, 0)),
            pl.BlockSpec((BN, 256), lambda g: (g, 0)),
        ],
        out_shape=[
            jax.ShapeDtypeStruct((N, 32), jnp.float32),
            jax.ShapeDtypeStruct((N, 256), jnp.float32),
            jax.ShapeDtypeStruct((N, 256), jnp.float32),
        ],
    )(S2, x1_o, Wroot2, b2, aux, G2, hg2, bg2, x1_o1, f3, Mab)


BB = 512
GRID_B = B_PAD // BB


def _k5_body(h_ref, mb1_ref, m2_ref, mb2_ref, m3_ref, mb3_ref, m4_ref,
             mb4_ref, log_ref):
    h = h_ref[...] + mb1_ref[...]
    e1 = jnp.where(h > 0, h, jnp.exp(h) - 1.0)
    t = jnp.dot(e1, m2_ref[...], preferred_element_type=jnp.float32) + mb2_ref[...]
    e2 = jnp.where(t > 0, t, jnp.exp(t) - 1.0)
    m34 = jnp.dot(m3_ref[...], m4_ref[...], preferred_element_type=jnp.float32)
    mb34 = jnp.dot(mb3_ref[...], m4_ref[...],
                   preferred_element_type=jnp.float32) + mb4_ref[...]
    log_ref[...] = jnp.dot(e2, m34, preferred_element_type=jnp.float32) + mb34


def _k5(H, mb1, M2, mb2, M3, mb3, M4, mb4):
    return pl.pallas_call(
        _k5_body,
        grid=(GRID_B,),
        in_specs=[
            pl.BlockSpec((BB, 256), lambda g: (g, 0)),
            pl.BlockSpec((1, 256), lambda g: (0, 0)),
            pl.BlockSpec((256, 128), lambda g: (0, 0)),
            pl.BlockSpec((1, 128), lambda g: (0, 0)),
            pl.BlockSpec((128, 65), lambda g: (0, 0)),
            pl.BlockSpec((1, 65), lambda g: (0, 0)),
            pl.BlockSpec((65, 2), lambda g: (0, 0)),
            pl.BlockSpec((1, 2), lambda g: (0, 0)),
        ],
        out_specs=pl.BlockSpec((BB, 2), lambda g: (g, 0)),
        out_shape=jax.ShapeDtypeStruct((B_PAD, 2), jnp.float32),
    )(H, mb1, M2, mb2, M3, mb3, M4, mb4)


def kernel(x, edge_index, edge_type, idx, attt, Wrel1, Wroot1, b1, Wrel2,
           Wroot2, b2, Wg1, bg1, Wg2, bg2, M1, mb1, M2, mb2, M3, mb3, M4,
           mb4, features3):
    src, dst = edge_index[0], edge_index[1]
    et = edge_type
    epad = E_PAD - E
    zpad = jnp.zeros((epad,), jnp.int32)
    tpad = jnp.full((epad,), N, jnp.int32)
    gidx = jnp.concatenate([et * N + src, zpad])
    sidx = jnp.concatenate([et * AR + dst, tpad])
    src_p = jnp.concatenate([src, zpad])
    dst_p = jnp.concatenate([dst, tpad])
    bpad = B_PAD - B
    aa_p = jnp.concatenate([idx[0], jnp.zeros((bpad,), jnp.int32)])
    bb_p = jnp.concatenate([idx[1], jnp.zeros((bpad,), jnp.int32)])

    a0, a1 = attt[0], attt[1]
    Ma = jnp.concatenate([a0 * M1[0:64], a1 * M1[64:96], a0 * M1[96:160],
                          a1 * M1[160:192], M1[192:704]], 0)
    Mb = jnp.concatenate([a0 * M1[704:768], a1 * M1[768:800], a0 * M1[800:864],
                          a1 * M1[864:896], M1[896:1408]], 0)
    Mab = jnp.stack([Ma, Mb])

    b1r = b1.reshape(1, 64)
    bg1r = bg1.reshape(1, 64)
    b2r = b2.reshape(1, 32)
    bg2r = bg2.reshape(1, 32)
    mb1r = mb1.reshape(1, 256)
    mb2r = mb2.reshape(1, 128)
    mb3r = mb3.reshape(1, 65)
    mb4r = mb4.reshape(1, 2)

    z80 = jnp.zeros((2 * AR // NS, 80), jnp.float32)
    z64 = jnp.zeros((A2 // NS, 64), jnp.float32)
    z32a = jnp.zeros((2 * AR // NS, 32), jnp.float32)
    z32b = jnp.zeros((A2 // NS, 32), jnp.float32)

    T1 = _k1(x, Wrel1)
    S1 = _sc_seg1(T1.reshape(2 * N, 80), gidx, sidx, z80).reshape(NC, 2, AR, 80)
    x1_o, hg1, T2, aux = _k2(x, S1, Wroot1, b1r, Wg1, Wrel2)
    G1 = _sc_seg2(hg1, src_p, dst_p, z64)
    S2 = _sc_seg3(T2.reshape(2 * N, 32), gidx, sidx, z32a).reshape(NC, 2, AR, 32)
    x1_o1, hg2 = _k3(G1, hg1, aux, bg1r, Wg2)
    G2 = _sc_seg4(hg2, src_p, dst_p, z32b)
    x2_o, Pa, Pb = _k4(S2, x1_o, Wroot2, b2r, aux, G2, hg2, bg2r, x1_o1,
                       features3, Mab)
    H = _sc_pair(Pa, Pb, aa_p, bb_p)
    logp = _k5(H, mb1r, M2, mb2r, M3, mb3r, M4, mb4r)
    return (logp[:B], x2_o)


# speedup vs baseline: 8.5652x; 1.1648x over previous
import functools

import jax
import jax.numpy as jnp
from jax import lax
from jax.experimental import pallas as pl
from jax.experimental.pallas import tpu as pltpu
from jax.experimental.pallas import tpu_sc as plsc

N = 10000
E = 320000
B = 100000

NC, NS = 2, 16
NW = NC * NS
CH = 128

CH5 = 64

E_PAD = ((E + 2 * NW * CH - 1) // (2 * NW * CH)) * (2 * NW * CH)
B_PAD = ((B + 2 * NW * CH5 - 1) // (2 * NW * CH5)) * (2 * NW * CH5)
AR = N + 48
A2 = N + 112

BN = 1000
GRID_N = N // BN


def _make_sc_segsum(table_rows, width, acc_rows, e_pad):
    epw = e_pad // NW
    n_ch = epw // CH
    stripe = acc_rows // NS
    rows_half = (e_pad // NC) // CH
    mesh = plsc.VectorSubcoreMesh(core_axis_name="c", subcore_axis_name="s")

    @functools.partial(
        pl.kernel, mesh=mesh,
        compiler_params=pltpu.CompilerParams(use_tc_tiling_on_sc=False),
        out_type=jax.ShapeDtypeStruct((NC, acc_rows, width), jnp.float32),
        scratch_types=[
            pltpu.VMEM((n_ch, CH), jnp.int32),
            pltpu.VMEM((n_ch, CH), jnp.int32),
            pltpu.VMEM((CH, width), jnp.float32),
            pltpu.VMEM((CH, width), jnp.float32),
            pltpu.VMEM_SHARED((acc_rows, width), jnp.float32),
            pltpu.SemaphoreType.DMA,
            pltpu.SemaphoreType.DMA,
        ],
    )
    def k(table, gidx, sidx, zeros, out, gv, sv, r0, r1, acc, s0, s1):
        c = lax.axis_index("c")
        s = lax.axis_index("s")
        pltpu.sync_copy(zeros, acc.at[pl.ds(s * stripe, stripe)])
        rb = c * rows_half + s * n_ch
        pltpu.sync_copy(gidx.at[pl.ds(rb, n_ch)], gv)
        pltpu.sync_copy(sidx.at[pl.ds(rb, n_ch)], sv)
        plsc.subcore_barrier()

        def fire(chunk, buf, sem):
            pltpu.async_copy(table.at[gv.at[chunk]], buf, sem)

        def wait(buf, sem):
            pltpu.make_async_copy(table.at[pl.ds(0, CH)], buf, sem).wait()

        def scat(buf, chunk):
            pltpu.sync_copy(buf, acc.at[sv.at[chunk]], add=True)

        fire(0, r0, s0)
        fire(1, r1, s1)

        def body(j2, carry):
            j = j2 * 2
            wait(r0, s0)
            scat(r0, j)
            fire(jnp.minimum(j + 2, n_ch - 1), r0, s0)
            wait(r1, s1)
            scat(r1, j + 1)
            fire(jnp.minimum(j + 3, n_ch - 1), r1, s1)
            return carry

        lax.fori_loop(0, n_ch // 2, body, 0)
        wait(r0, s0)
        wait(r1, s1)
        plsc.subcore_barrier()
        pltpu.sync_copy(acc.at[pl.ds(s * stripe, stripe)],
                        out.at[c, pl.ds(s * stripe, stripe)])

    return k


def _make_sc_pair_gather(width):
    ppw = B_PAD // NW
    n_ch = ppw // CH5
    kk = width // 16
    mesh = plsc.VectorSubcoreMesh(core_axis_name="c", subcore_axis_name="s")

    @functools.partial(
        pl.kernel, mesh=mesh,
        compiler_params=pltpu.CompilerParams(use_tc_tiling_on_sc=False),
        out_type=jax.ShapeDtypeStruct((B_PAD, width), jnp.float32),
        scratch_types=[
            pltpu.VMEM((n_ch, CH5), jnp.int32),
            pltpu.VMEM((n_ch, CH5), jnp.int32),
            pltpu.VMEM((CH5, width), jnp.float32),
            pltpu.VMEM((CH5, width), jnp.float32),
            pltpu.VMEM((CH5, width), jnp.float32),
            pltpu.VMEM((CH5, width), jnp.float32),
            pltpu.SemaphoreType.DMA,
            pltpu.SemaphoreType.DMA,
            pltpu.SemaphoreType.DMA,
            pltpu.SemaphoreType.DMA,
        ],
    )
    def k(pa, pb, aidx, bidx, out, av, bv, ra0, rb0, ra1, rb1,
          sa0, sb0, sa1, sb1):
        c = lax.axis_index("c")
        s = lax.axis_index("s")
        w = s * NC + c
        base = w * ppw
        pltpu.sync_copy(aidx.at[pl.ds(w * n_ch, n_ch)], av)
        pltpu.sync_copy(bidx.at[pl.ds(w * n_ch, n_ch)], bv)

        def fire(chunk, ra, rb, sa, sb):
            pltpu.async_copy(pa.at[av.at[chunk]], ra, sa)
            pltpu.async_copy(pb.at[bv.at[chunk]], rb, sb)

        def wait(ra, rb, sa, sb):
            pltpu.make_async_copy(pa.at[pl.ds(0, CH5)], ra, sa).wait()
            pltpu.make_async_copy(pb.at[pl.ds(0, CH5)], rb, sb).wait()

        def process(ra, rb, chunk):
            def add_row(r, carry2):
                for q in range(kk):
                    ra[r, pl.ds(q * 16, 16)] = (
                        ra[r, pl.ds(q * 16, 16)] + rb[r, pl.ds(q * 16, 16)])
                return carry2

            lax.fori_loop(0, CH5, add_row, 0)
            pltpu.sync_copy(ra, out.at[pl.ds(base + chunk * CH5, CH5)])

        fire(0, ra0, rb0, sa0, sb0)
        fire(1, ra1, rb1, sa1, sb1)

        def body(j2, carry):
            j = j2 * 2
            wait(ra0, rb0, sa0, sb0)
            process(ra0, rb0, j)
            fire(jnp.minimum(j + 2, n_ch - 1), ra0, rb0, sa0, sb0)
            wait(ra1, rb1, sa1, sb1)
            process(ra1, rb1, j + 1)
            fire(jnp.minimum(j + 3, n_ch - 1), ra1, rb1, sa1, sb1)
            return carry

        lax.fori_loop(0, n_ch // 2, body, 0)
        wait(ra0, rb0, sa0, sb0)
        wait(ra1, rb1, sa1, sb1)

    return k


def _make_sc_count():
    epw = E_PAD // NW
    n_ch = epw // CH
    acc_rows = 2 * AR
    stripe = acc_rows // NS
    rows_half = (E_PAD // NC) // CH
    mesh = plsc.VectorSubcoreMesh(core_axis_name="c", subcore_axis_name="s")

    @functools.partial(
        pl.kernel, mesh=mesh,
        compiler_params=pltpu.CompilerParams(use_tc_tiling_on_sc=False),
        out_type=jax.ShapeDtypeStruct((NC, acc_rows, 16), jnp.float32),
        scratch_types=[
            pltpu.VMEM((n_ch, CH), jnp.int32),
            pltpu.VMEM((CH, 16), jnp.float32),
            pltpu.VMEM_SHARED((acc_rows, 16), jnp.float32),
        ],
    )
    def k(sidx, ones, zeros, out, sv, ov, acc):
        c = lax.axis_index("c")
        s = lax.axis_index("s")
        pltpu.sync_copy(zeros, acc.at[pl.ds(s * stripe, stripe)])
        pltpu.sync_copy(ones, ov)
        rb = c * rows_half + s * n_ch
        pltpu.sync_copy(sidx.at[pl.ds(rb, n_ch)], sv)
        plsc.subcore_barrier()

        def body(j, carry):
            pltpu.sync_copy(ov, acc.at[sv.at[j]], add=True)
            return carry

        lax.fori_loop(0, n_ch, body, 0)
        plsc.subcore_barrier()
        pltpu.sync_copy(acc.at[pl.ds(s * stripe, stripe)],
                        out.at[c, pl.ds(s * stripe, stripe)])

    return k


_sc_count = _make_sc_count()
_sc_seg1 = _make_sc_segsum(2 * N, 64, 2 * AR, E_PAD)
_sc_seg2 = _make_sc_segsum(N, 64, A2, E_PAD)
_sc_seg3 = _make_sc_segsum(2 * N, 32, 2 * AR, E_PAD)
_sc_seg4 = _make_sc_segsum(N, 32, A2, E_PAD)
_sc_pair = _make_sc_pair_gather(256)


def _k1_body(x_ref, w_ref, t_ref):
    xb = x_ref[...]
    t_ref[0] = jnp.dot(xb, w_ref[0], preferred_element_type=jnp.float32)
    t_ref[1] = jnp.dot(xb, w_ref[1], preferred_element_type=jnp.float32)


def _k1(x, Wrel1):
    return pl.pallas_call(
        _k1_body,
        grid=(GRID_N,),
        in_specs=[
            pl.BlockSpec((BN, 128), lambda g: (g, 0)),
            pl.BlockSpec((2, 128, 64), lambda g: (0, 0, 0)),
        ],
        out_specs=pl.BlockSpec((2, BN, 64), lambda g: (0, g, 0)),
        out_shape=jax.ShapeDtypeStruct((2, N, 64), jnp.float32),
    )(x, Wrel1)


def _k2_body(x_ref, s1_ref, cnt_ref, wr_ref, b1_ref, wg_ref, w2_ref,
             x1_ref, hg1_ref, t2_ref, aux_ref):
    S0 = s1_ref[0, 0] + s1_ref[1, 0]
    S1 = s1_ref[0, 1] + s1_ref[1, 1]
    c0 = cnt_ref[0, 0, :, 0] + cnt_ref[1, 0, :, 0]
    c1 = cnt_ref[0, 1, :, 0] + cnt_ref[1, 1, :, 0]
    dinv = lax.rsqrt(c0 + c1 + 1.0)
    ic0 = 1.0 / jnp.maximum(c0, 1.0)
    ic1 = 1.0 / jnp.maximum(c1, 1.0)
    xb = x_ref[...]
    x1 = jnp.dot(xb, wr_ref[...], preferred_element_type=jnp.float32) + b1_ref[...]
    x1 = x1 + S0 * ic0[:, None] + S1 * ic1[:, None]
    x1 = jnp.maximum(x1, 0.0)
    x1_ref[...] = x1
    hg1_ref[...] = dinv[:, None] * jnp.dot(xb, wg_ref[...],
                                           preferred_element_type=jnp.float32)
    t2_ref[0] = jnp.dot(x1, w2_ref[0], preferred_element_type=jnp.float32)
    t2_ref[1] = jnp.dot(x1, w2_ref[1], preferred_element_type=jnp.float32)
    aux_ref[...] = jnp.concatenate(
        [dinv[:, None], ic0[:, None], ic1[:, None],
         jnp.zeros((BN, 5), jnp.float32)], -1)


def _k2(x, S1, CNT, Wroot1, b1, Wg1, Wrel2):
    return pl.pallas_call(
        _k2_body,
        grid=(GRID_N,),
        in_specs=[
            pl.BlockSpec((BN, 128), lambda g: (g, 0)),
            pl.BlockSpec((2, 2, BN, 64), lambda g: (0, 0, g, 0)),
            pl.BlockSpec((2, 2, BN, 16), lambda g: (0, 0, g, 0)),
            pl.BlockSpec((128, 64), lambda g: (0, 0)),
            pl.BlockSpec((1, 64), lambda g: (0, 0)),
            pl.BlockSpec((128, 64), lambda g: (0, 0)),
            pl.BlockSpec((2, 64, 32), lambda g: (0, 0, 0)),
        ],
        out_specs=[
            pl.BlockSpec((BN, 64), lambda g: (g, 0)),
            pl.BlockSpec((BN, 64), lambda g: (g, 0)),
            pl.BlockSpec((2, BN, 32), lambda g: (0, g, 0)),
            pl.BlockSpec((BN, 8), lambda g: (g, 0)),
        ],
        out_shape=[
            jax.ShapeDtypeStruct((N, 64), jnp.float32),
            jax.ShapeDtypeStruct((N, 64), jnp.float32),
            jax.ShapeDtypeStruct((2, N, 32), jnp.float32),
            jax.ShapeDtypeStruct((N, 8), jnp.float32),
        ],
    )(x, S1, CNT, Wroot1, b1, Wg1, Wrel2)


def _k3_body(g1_ref, hg1_ref, aux_ref, bg1_ref, wg2_ref, x11_ref, hg2_ref):
    dinv = aux_ref[...][:, 0]
    G = g1_ref[0] + g1_ref[1]
    x11 = jnp.maximum(dinv[:, None] * (G + hg1_ref[...]) + bg1_ref[...], 0.0)
    x11_ref[...] = x11
    hg2_ref[...] = dinv[:, None] * jnp.dot(x11, wg2_ref[...],
                                           preferred_element_type=jnp.float32)


def _k3(G1, hg1, aux, bg1, Wg2):
    return pl.pallas_call(
        _k3_body,
        grid=(GRID_N,),
        in_specs=[
            pl.BlockSpec((2, BN, 64), lambda g: (0, g, 0)),
            pl.BlockSpec((BN, 64), lambda g: (g, 0)),
            pl.BlockSpec((BN, 8), lambda g: (g, 0)),
            pl.BlockSpec((1, 64), lambda g: (0, 0)),
            pl.BlockSpec((64, 32), lambda g: (0, 0)),
        ],
        out_specs=[
            pl.BlockSpec((BN, 64), lambda g: (g, 0)),
            pl.BlockSpec((BN, 32), lambda g: (g, 0)),
        ],
        out_shape=[
            jax.ShapeDtypeStruct((N, 64), jnp.float32),
            jax.ShapeDtypeStruct((N, 32), jnp.float32),
        ],
    )(G1, hg1, aux, bg1, Wg2)


def _k4_body(s2_ref, x1_ref, wr2_ref, b2_ref, aux_ref, g2_ref, hg2_ref,
             bg2_ref, x11_ref, f3_ref, mab_ref, x2_ref, pa_ref, pb_ref):
    aux = aux_ref[...]
    dinv = aux[:, 0]
    ic0 = aux[:, 1]
    ic1 = aux[:, 2]
    S0 = s2_ref[0, 0] + s2_ref[1, 0]
    S1 = s2_ref[0, 1] + s2_ref[1, 1]
    x1 = x1_ref[...]
    x2 = jnp.dot(x1, wr2_ref[...], preferred_element_type=jnp.float32) + b2_ref[...]
    x2 = x2 + S0 * ic0[:, None] + S1 * ic1[:, None]
    x2_ref[...] = x2
    G2 = g2_ref[0] + g2_ref[1]
    x21 = dinv[:, None] * (G2 + hg2_ref[...]) + bg2_ref[...]
    Z = jnp.concatenate([x11_ref[...], x21, x1, x2, f3_ref[...]], -1)
    pa_ref[...] = jnp.dot(Z, mab_ref[0], preferred_element_type=jnp.float32)
    pb_ref[...] = jnp.dot(Z, mab_ref[1], preferred_element_type=jnp.float32)


def _k4(S2, x1_o, Wroot2, b2, aux, G2, hg2, bg2, x1_o1, f3, Mab):
    return pl.pallas_call(
        _k4_body,
        grid=(GRID_N,),
        in_specs=[
            pl.BlockSpec((2, 2, BN, 32), lambda g: (0, 0, g, 0)),
            pl.BlockSpec((BN, 64), lambda g: (g, 0)),
            pl.BlockSpec((64, 32), lambda g: (0, 0)),
            pl.BlockSpec((1, 32), lambda g: (0, 0)),
            pl.BlockSpec((BN, 8), lambda g: (g, 0)),
            pl.BlockSpec((2, BN, 32), lambda g: (0, g, 0)),
            pl.BlockSpec((BN, 32), lambda g: (g, 0)),
            pl.BlockSpec((1, 32), lambda g: (0, 0)),
            pl.BlockSpec((BN, 64), lambda g: (g, 0)),
            pl.BlockSpec((BN, 512), lambda g: (g, 0)),
            pl.BlockSpec((2, 704, 256), lambda g: (0, 0, 0)),
        ],
        out_specs=[
            pl.BlockSpec((BN, 32), lambda g: (g, 0)),
            pl.BlockSpec((BN, 256), lambda g: (g, 0)),
            pl.BlockSpec((BN, 256), lambda g: (g, 0)),
        ],
        out_shape=[
            jax.ShapeDtypeStruct((N, 32), jnp.float32),
            jax.ShapeDtypeStruct((N, 256), jnp.float32),
            jax.ShapeDtypeStruct((N, 256), jnp.float32),
        ],
    )(S2, x1_o, Wroot2, b2, aux, G2, hg2, bg2, x1_o1, f3, Mab)


BB = 512
GRID_B = B_PAD // BB


def _k5_body(h_ref, mb1_ref, m2_ref, mb2_ref, m3_ref, mb3_ref, m4_ref,
             mb4_ref, log_ref):
    h = h_ref[...] + mb1_ref[...]
    e1 = jnp.where(h > 0, h, jnp.exp(h) - 1.0)
    t = jnp.dot(e1, m2_ref[...], preferred_element_type=jnp.float32) + mb2_ref[...]
    e2 = jnp.where(t > 0, t, jnp.exp(t) - 1.0)
    m34 = jnp.dot(m3_ref[...], m4_ref[...], preferred_element_type=jnp.float32)
    mb34 = jnp.dot(mb3_ref[...], m4_ref[...],
                   preferred_element_type=jnp.float32) + mb4_ref[...]
    log_ref[...] = jnp.dot(e2, m34, preferred_element_type=jnp.float32) + mb34


def _k5(H, mb1, M2, mb2, M3, mb3, M4, mb4):
    return pl.pallas_call(
        _k5_body,
        grid=(GRID_B,),
        in_specs=[
            pl.BlockSpec((BB, 256), lambda g: (g, 0)),
            pl.BlockSpec((1, 256), lambda g: (0, 0)),
            pl.BlockSpec((256, 128), lambda g: (0, 0)),
            pl.BlockSpec((1, 128), lambda g: (0, 0)),
            pl.BlockSpec((128, 65), lambda g: (0, 0)),
            pl.BlockSpec((1, 65), lambda g: (0, 0)),
            pl.BlockSpec((65, 2), lambda g: (0, 0)),
            pl.BlockSpec((1, 2), lambda g: (0, 0)),
        ],
        out_specs=pl.BlockSpec((BB, 2), lambda g: (g, 0)),
        out_shape=jax.ShapeDtypeStruct((B_PAD, 2), jnp.float32),
    )(H, mb1, M2, mb2, M3, mb3, M4, mb4)


def kernel(x, edge_index, edge_type, idx, attt, Wrel1, Wroot1, b1, Wrel2,
           Wroot2, b2, Wg1, bg1, Wg2, bg2, M1, mb1, M2, mb2, M3, mb3, M4,
           mb4, features3):
    src, dst = edge_index[0], edge_index[1]
    et = edge_type
    epad = E_PAD - E
    zpad = jnp.zeros((epad,), jnp.int32)
    tpad = jnp.full((epad,), N, jnp.int32)
    gidx = jnp.concatenate([et * N + src, zpad]).reshape(-1, CH)
    sidx = jnp.concatenate([et * AR + dst, tpad]).reshape(-1, CH)
    src_p = jnp.concatenate([src, zpad]).reshape(-1, CH)
    dst_p = jnp.concatenate([dst, tpad]).reshape(-1, CH)
    bpad = B_PAD - B
    aa_p = jnp.concatenate([idx[0], jnp.zeros((bpad,), jnp.int32)]).reshape(-1, CH5)
    bb_p = jnp.concatenate([idx[1], jnp.zeros((bpad,), jnp.int32)]).reshape(-1, CH5)

    a0, a1 = attt[0], attt[1]
    Ma = jnp.concatenate([a0 * M1[0:64], a1 * M1[64:96], a0 * M1[96:160],
                          a1 * M1[160:192], M1[192:704]], 0)
    Mb = jnp.concatenate([a0 * M1[704:768], a1 * M1[768:800], a0 * M1[800:864],
                          a1 * M1[864:896], M1[896:1408]], 0)
    Mab = jnp.stack([Ma, Mb])

    b1r = b1.reshape(1, 64)
    bg1r = bg1.reshape(1, 64)
    b2r = b2.reshape(1, 32)
    bg2r = bg2.reshape(1, 32)
    mb1r = mb1.reshape(1, 256)
    mb2r = mb2.reshape(1, 128)
    mb3r = mb3.reshape(1, 65)
    mb4r = mb4.reshape(1, 2)

    z16 = jnp.zeros((2 * AR // NS, 16), jnp.float32)
    z64a = jnp.zeros((2 * AR // NS, 64), jnp.float32)
    z64 = jnp.zeros((A2 // NS, 64), jnp.float32)
    z32a = jnp.zeros((2 * AR // NS, 32), jnp.float32)
    z32b = jnp.zeros((A2 // NS, 32), jnp.float32)
    ones16 = jnp.ones((CH, 16), jnp.float32)

    CNT = _sc_count(sidx, ones16, z16).reshape(NC, 2, AR, 16)
    T1 = _k1(x, Wrel1)
    S1 = _sc_seg1(T1.reshape(2 * N, 64), gidx, sidx, z64a).reshape(NC, 2, AR, 64)
    x1_o, hg1, T2, aux = _k2(x, S1, CNT, Wroot1, b1r, Wg1, Wrel2)
    G1 = _sc_seg2(hg1, src_p, dst_p, z64)
    S2 = _sc_seg3(T2.reshape(2 * N, 32), gidx, sidx, z32a).reshape(NC, 2, AR, 32)
    x1_o1, hg2 = _k3(G1, hg1, aux, bg1r, Wg2)
    G2 = _sc_seg4(hg2, src_p, dst_p, z32b)
    x2_o, Pa, Pb = _k4(S2, x1_o, Wroot2, b2r, aux, G2, hg2, bg2r, x1_o1,
                       features3, Mab)
    H = _sc_pair(Pa, Pb, aa_p, bb_p)
    logp = _k5(H, mb1r, M2, mb2r, M3, mb3r, M4, mb4r)
    return (logp[:B], x2_o)


# speedup vs baseline: 9.7523x; 1.1386x over previous
import functools

import jax
import jax.numpy as jnp
from jax import lax
from jax.experimental import pallas as pl
from jax.experimental.pallas import tpu as pltpu
from jax.experimental.pallas import tpu_sc as plsc

N = 10000
E = 320000
B = 100000

NC, NS = 2, 16
NW = NC * NS
CH = 128

CH5 = 64

E_PAD = ((E + 2 * NW * CH - 1) // (2 * NW * CH)) * (2 * NW * CH)
B_PAD = ((B + 2 * NW * CH5 - 1) // (2 * NW * CH5)) * (2 * NW * CH5)

EC0 = 112
EC1 = 48
E_ROWS_PAD = 17 * EC0 + 15 * EC1
PC0 = 66
PC1 = 34
P_IDX_PAD = (17 * PC0 + 15 * PC1) * CH5
AR = N + 48
A2 = N + 112

BN = 1000
GRID_N = N // BN


def _make_sc_segsum(table_rows, width, acc_rows, e_pad):
    del e_pad
    stripe = acc_rows // NS
    mesh = plsc.VectorSubcoreMesh(core_axis_name="c", subcore_axis_name="s")

    @functools.partial(
        pl.kernel, mesh=mesh,
        compiler_params=pltpu.CompilerParams(use_tc_tiling_on_sc=False),
        out_type=jax.ShapeDtypeStruct((NC, acc_rows, width), jnp.float32),
        scratch_types=[
            pltpu.VMEM((EC0, CH), jnp.int32),
            pltpu.VMEM((EC0, CH), jnp.int32),
            pltpu.VMEM((CH, width), jnp.float32),
            pltpu.VMEM((CH, width), jnp.float32),
            pltpu.VMEM_SHARED((acc_rows, width), jnp.float32),
            pltpu.SemaphoreType.DMA,
            pltpu.SemaphoreType.DMA,
        ],
    )
    def k(table, gidx, sidx, zeros, out, gv, sv, r0, r1, acc, s0, s1):
        c = lax.axis_index("c")
        s = lax.axis_index("s")
        pltpu.sync_copy(zeros, acc.at[pl.ds(s * stripe, stripe)])
        n_ch = jnp.where(c == 0, EC0, EC1)
        rb = jnp.where(c == 0, s * EC0, NS * EC0 + s * EC1)
        pltpu.sync_copy(gidx.at[pl.ds(rb, EC0)], gv)
        pltpu.sync_copy(sidx.at[pl.ds(rb, EC0)], sv)
        plsc.subcore_barrier()

        def fire(chunk, buf, sem):
            pltpu.async_copy(table.at[gv.at[chunk]], buf, sem)

        def wait(buf, sem):
            pltpu.make_async_copy(table.at[pl.ds(0, CH)], buf, sem).wait()

        def scat(buf, chunk):
            pltpu.sync_copy(buf, acc.at[sv.at[chunk]], add=True)

        fire(0, r0, s0)
        fire(1, r1, s1)

        def body(j2, carry):
            j = j2 * 2
            wait(r0, s0)
            scat(r0, j)
            fire(jnp.minimum(j + 2, n_ch - 1), r0, s0)
            wait(r1, s1)
            scat(r1, j + 1)
            fire(jnp.minimum(j + 3, n_ch - 1), r1, s1)
            return carry

        lax.fori_loop(0, n_ch // 2, body, 0)
        wait(r0, s0)
        wait(r1, s1)
        plsc.subcore_barrier()
        pltpu.sync_copy(acc.at[pl.ds(s * stripe, stripe)],
                        out.at[c, pl.ds(s * stripe, stripe)])

    return k


def _make_sc_pair_gather(width):
    kk = width // 16
    mesh = plsc.VectorSubcoreMesh(core_axis_name="c", subcore_axis_name="s")

    @functools.partial(
        pl.kernel, mesh=mesh,
        compiler_params=pltpu.CompilerParams(use_tc_tiling_on_sc=True),
        out_type=jax.ShapeDtypeStruct((B_PAD, width), jnp.float32),
        scratch_types=[
            pltpu.VMEM((PC0 * CH5,), jnp.int32),
            pltpu.VMEM((PC0 * CH5,), jnp.int32),
            pltpu.VMEM((CH5, width), jnp.float32),
            pltpu.VMEM((CH5, width), jnp.float32),
            pltpu.VMEM((CH5, width), jnp.float32),
            pltpu.VMEM((CH5, width), jnp.float32),
            pltpu.SemaphoreType.DMA,
            pltpu.SemaphoreType.DMA,
            pltpu.SemaphoreType.DMA,
            pltpu.SemaphoreType.DMA,
        ],
    )
    def k(pa, pb, aidx, bidx, out, av, bv, ra0, rb0, ra1, rb1,
          sa0, sb0, sa1, sb1):
        c = lax.axis_index("c")
        s = lax.axis_index("s")
        n_ch = jnp.where(c == 0, PC0, PC1)
        cb = jnp.where(c == 0, s * PC0, NS * PC0 + s * PC1)
        base = cb * CH5
        pltpu.sync_copy(aidx.at[pl.ds(base, PC0 * CH5)], av)
        pltpu.sync_copy(bidx.at[pl.ds(base, PC0 * CH5)], bv)

        def fire(chunk, ra, rb, sa, sb):
            pltpu.async_copy(pa.at[av.at[pl.ds(chunk * CH5, CH5)]], ra, sa)
            pltpu.async_copy(pb.at[bv.at[pl.ds(chunk * CH5, CH5)]], rb, sb)

        def wait(ra, rb, sa, sb):
            pltpu.make_async_copy(pa.at[pl.ds(0, CH5)], ra, sa).wait()
            pltpu.make_async_copy(pb.at[pl.ds(0, CH5)], rb, sb).wait()

        def process(ra, rb, chunk):
            def add_row(r, carry2):
                for q in range(kk):
                    ra[r, pl.ds(q * 16, 16)] = (
                        ra[r, pl.ds(q * 16, 16)] + rb[r, pl.ds(q * 16, 16)])
                return carry2

            lax.fori_loop(0, CH5, add_row, 0)
            pltpu.sync_copy(ra, out.at[pl.ds(base + chunk * CH5, CH5)])

        fire(0, ra0, rb0, sa0, sb0)
        fire(1, ra1, rb1, sa1, sb1)

        def body(j2, carry):
            j = j2 * 2
            wait(ra0, rb0, sa0, sb0)
            process(ra0, rb0, j)
            fire(jnp.minimum(j + 2, n_ch - 1), ra0, rb0, sa0, sb0)
            wait(ra1, rb1, sa1, sb1)
            process(ra1, rb1, j + 1)
            fire(jnp.minimum(j + 3, n_ch - 1), ra1, rb1, sa1, sb1)
            return carry

        lax.fori_loop(0, n_ch // 2, body, 0)
        wait(ra0, rb0, sa0, sb0)
        wait(ra1, rb1, sa1, sb1)

    return k


def _make_sc_count():
    acc_rows = 2 * AR
    stripe = acc_rows // NS
    mesh = plsc.VectorSubcoreMesh(core_axis_name="c", subcore_axis_name="s")

    @functools.partial(
        pl.kernel, mesh=mesh,
        compiler_params=pltpu.CompilerParams(use_tc_tiling_on_sc=False),
        out_type=jax.ShapeDtypeStruct((NC, acc_rows, 16), jnp.float32),
        scratch_types=[
            pltpu.VMEM((EC0, CH), jnp.int32),
            pltpu.VMEM((CH, 16), jnp.float32),
            pltpu.VMEM_SHARED((acc_rows, 16), jnp.float32),
        ],
    )
    def k(sidx, ones, zeros, out, sv, ov, acc):
        c = lax.axis_index("c")
        s = lax.axis_index("s")
        pltpu.sync_copy(zeros, acc.at[pl.ds(s * stripe, stripe)])
        pltpu.sync_copy(ones, ov)
        n_ch = jnp.where(c == 0, EC0, EC1)
        rb = jnp.where(c == 0, s * EC0, NS * EC0 + s * EC1)
        pltpu.sync_copy(sidx.at[pl.ds(rb, EC0)], sv)
        plsc.subcore_barrier()

        def body(j, carry):
            pltpu.sync_copy(ov, acc.at[sv.at[j]], add=True)
            return carry

        lax.fori_loop(0, n_ch, body, 0)
        plsc.subcore_barrier()
        pltpu.sync_copy(acc.at[pl.ds(s * stripe, stripe)],
                        out.at[c, pl.ds(s * stripe, stripe)])

    return k


_sc_count = _make_sc_count()
_sc_seg1 = _make_sc_segsum(2 * N, 64, 2 * AR, E_PAD)
_sc_seg2 = _make_sc_segsum(N, 64, A2, E_PAD)
_sc_seg3 = _make_sc_segsum(2 * N, 32, 2 * AR, E_PAD)
_sc_seg4 = _make_sc_segsum(N, 32, A2, E_PAD)
_sc_pair = _make_sc_pair_gather(256)


def _k1_body(x_ref, w_ref, t_ref):
    xb = x_ref[...]
    t_ref[0] = jnp.dot(xb, w_ref[0], preferred_element_type=jnp.float32)
    t_ref[1] = jnp.dot(xb, w_ref[1], preferred_element_type=jnp.float32)


def _k1(x, Wrel1):
    return pl.pallas_call(
        _k1_body,
        grid=(GRID_N,),
        in_specs=[
            pl.BlockSpec((BN, 128), lambda g: (g, 0)),
            pl.BlockSpec((2, 128, 64), lambda g: (0, 0, 0)),
        ],
        out_specs=pl.BlockSpec((2, BN, 64), lambda g: (0, g, 0)),
        out_shape=jax.ShapeDtypeStruct((2, N, 64), jnp.float32),
    )(x, Wrel1)


def _k2_body(x_ref, s1_ref, cnt_ref, wr_ref, b1_ref, wg_ref, w2_ref,
             x1_ref, hg1_ref, t2_ref, aux_ref):
    S0 = s1_ref[0, 0] + s1_ref[1, 0]
    S1 = s1_ref[0, 1] + s1_ref[1, 1]
    c0 = cnt_ref[0, 0, :, 0] + cnt_ref[1, 0, :, 0]
    c1 = cnt_ref[0, 1, :, 0] + cnt_ref[1, 1, :, 0]
    dinv = lax.rsqrt(c0 + c1 + 1.0)
    ic0 = 1.0 / jnp.maximum(c0, 1.0)
    ic1 = 1.0 / jnp.maximum(c1, 1.0)
    xb = x_ref[...]
    x1 = jnp.dot(xb, wr_ref[...], preferred_element_type=jnp.float32) + b1_ref[...]
    x1 = x1 + S0 * ic0[:, None] + S1 * ic1[:, None]
    x1 = jnp.maximum(x1, 0.0)
    x1_ref[...] = x1
    hg1_ref[...] = dinv[:, None] * jnp.dot(xb, wg_ref[...],
                                           preferred_element_type=jnp.float32)
    t2_ref[0] = jnp.dot(x1, w2_ref[0], preferred_element_type=jnp.float32)
    t2_ref[1] = jnp.dot(x1, w2_ref[1], preferred_element_type=jnp.float32)
    aux_ref[...] = jnp.concatenate(
        [dinv[:, None], ic0[:, None], ic1[:, None],
         jnp.zeros((BN, 5), jnp.float32)], -1)


def _k2(x, S1, CNT, Wroot1, b1, Wg1, Wrel2):
    return pl.pallas_call(
        _k2_body,
        grid=(GRID_N,),
        in_specs=[
            pl.BlockSpec((BN, 128), lambda g: (g, 0)),
            pl.BlockSpec((2, 2, BN, 64), lambda g: (0, 0, g, 0)),
            pl.BlockSpec((2, 2, BN, 16), lambda g: (0, 0, g, 0)),
            pl.BlockSpec((128, 64), lambda g: (0, 0)),
            pl.BlockSpec((1, 64), lambda g: (0, 0)),
            pl.BlockSpec((128, 64), lambda g: (0, 0)),
            pl.BlockSpec((2, 64, 32), lambda g: (0, 0, 0)),
        ],
        out_specs=[
            pl.BlockSpec((BN, 64), lambda g: (g, 0)),
            pl.BlockSpec((BN, 64), lambda g: (g, 0)),
            pl.BlockSpec((2, BN, 32), lambda g: (0, g, 0)),
            pl.BlockSpec((BN, 8), lambda g: (g, 0)),
        ],
        out_shape=[
            jax.ShapeDtypeStruct((N, 64), jnp.float32),
            jax.ShapeDtypeStruct((N, 64), jnp.float32),
            jax.ShapeDtypeStruct((2, N, 32), jnp.float32),
            jax.ShapeDtypeStruct((N, 8), jnp.float32),
        ],
    )(x, S1, CNT, Wroot1, b1, Wg1, Wrel2)


def _k3_body(g1_ref, hg1_ref, aux_ref, bg1_ref, wg2_ref, x11_ref, hg2_ref):
    dinv = aux_ref[...][:, 0]
    G = g1_ref[0] + g1_ref[1]
    x11 = jnp.maximum(dinv[:, None] * (G + hg1_ref[...]) + bg1_ref[...], 0.0)
    x11_ref[...] = x11
    hg2_ref[...] = dinv[:, None] * jnp.dot(x11, wg2_ref[...],
                                           preferred_element_type=jnp.float32)


def _k3(G1, hg1, aux, bg1, Wg2):
    return pl.pallas_call(
        _k3_body,
        grid=(GRID_N,),
        in_specs=[
            pl.BlockSpec((2, BN, 64), lambda g: (0, g, 0)),
            pl.BlockSpec((BN, 64), lambda g: (g, 0)),
            pl.BlockSpec((BN, 8), lambda g: (g, 0)),
            pl.BlockSpec((1, 64), lambda g: (0, 0)),
            pl.BlockSpec((64, 32), lambda g: (0, 0)),
        ],
        out_specs=[
            pl.BlockSpec((BN, 64), lambda g: (g, 0)),
            pl.BlockSpec((BN, 32), lambda g: (g, 0)),
        ],
        out_shape=[
            jax.ShapeDtypeStruct((N, 64), jnp.float32),
            jax.ShapeDtypeStruct((N, 32), jnp.float32),
        ],
    )(G1, hg1, aux, bg1, Wg2)


def _k4_body(s2_ref, x1_ref, wr2_ref, b2_ref, aux_ref, g2_ref, hg2_ref,
             bg2_ref, x11_ref, f3_ref, mab_ref, x2_ref, pa_ref, pb_ref):
    aux = aux_ref[...]
    dinv = aux[:, 0]
    ic0 = aux[:, 1]
    ic1 = aux[:, 2]
    S0 = s2_ref[0, 0] + s2_ref[1, 0]
    S1 = s2_ref[0, 1] + s2_ref[1, 1]
    x1 = x1_ref[...]
    x2 = jnp.dot(x1, wr2_ref[...], preferred_element_type=jnp.float32) + b2_ref[...]
    x2 = x2 + S0 * ic0[:, None] + S1 * ic1[:, None]
    x2_ref[...] = x2
    G2 = g2_ref[0] + g2_ref[1]
    x21 = dinv[:, None] * (G2 + hg2_ref[...]) + bg2_ref[...]
    Z = jnp.concatenate([x11_ref[...], x21, x1, x2, f3_ref[...]], -1)
    pa_ref[...] = jnp.dot(Z, mab_ref[0], preferred_element_type=jnp.float32)
    pb_ref[...] = jnp.dot(Z, mab_ref[1], preferred_element_type=jnp.float32)


def _k4(S2, x1_o, Wroot2, b2, aux, G2, hg2, bg2, x1_o1, f3, Mab):
    return pl.pallas_call(
        _k4_body,
        grid=(GRID_N,),
        in_specs=[
            pl.BlockSpec((2, 2, BN, 32), lambda g: (0, 0, g, 0)),
            pl.BlockSpec((BN, 64), lambda g: (g, 0)),
            pl.BlockSpec((64, 32), lambda g: (0, 0)),
            pl.BlockSpec((1, 32), lambda g: (0, 0)),
            pl.BlockSpec((BN, 8), lambda g: (g, 0)),
            pl.BlockSpec((2, BN, 32), lambda g: (0, g, 0)),
            pl.BlockSpec((BN, 32), lambda g: (g, 0)),
            pl.BlockSpec((1, 32), lambda g: (0, 0)),
            pl.BlockSpec((BN, 64), lambda g: (g, 0)),
            pl.BlockSpec((BN, 512), lambda g: (g, 0)),
            pl.BlockSpec((2, 704, 256), lambda g: (0, 0, 0)),
        ],
        out_specs=[
            pl.BlockSpec((BN, 32), lambda g: (g, 0)),
            pl.BlockSpec((BN, 256), lambda g: (g, 0)),
            pl.BlockSpec((BN, 256), lambda g: (g, 0)),
        ],
        out_shape=[
            jax.ShapeDtypeStruct((N, 32), jnp.float32),
            jax.ShapeDtypeStruct((N, 256), jnp.float32),
            jax.ShapeDtypeStruct((N, 256), jnp.float32),
        ],
    )(S2, x1_o, Wroot2, b2, aux, G2, hg2, bg2, x1_o1, f3, Mab)


BB = 1000
GRID_B = B // BB


def _k5_body(h_ref, mb1_ref, m2_ref, mb2_ref, m3_ref, mb3_ref, m4_ref,
             mb4_ref, log_ref):
    h = h_ref[...] + mb1_ref[...]
    e1 = jnp.where(h > 0, h, jnp.exp(h) - 1.0)
    t = jnp.dot(e1, m2_ref[...], preferred_element_type=jnp.float32) + mb2_ref[...]
    e2 = jnp.where(t > 0, t, jnp.exp(t) - 1.0)
    m34 = jnp.dot(m3_ref[...], m4_ref[...], preferred_element_type=jnp.float32)
    mb34 = jnp.dot(mb3_ref[...], m4_ref[...],
                   preferred_element_type=jnp.float32) + mb4_ref[...]
    log_ref[...] = jnp.dot(e2, m34, preferred_element_type=jnp.float32) + mb34


def _k5(H, mb1, M2, mb2, M3, mb3, M4, mb4):
    return pl.pallas_call(
        _k5_body,
        grid=(GRID_B,),
        in_specs=[
            pl.BlockSpec((BB, 256), lambda g: (g, 0)),
            pl.BlockSpec((1, 256), lambda g: (0, 0)),
            pl.BlockSpec((256, 128), lambda g: (0, 0)),
            pl.BlockSpec((1, 128), lambda g: (0, 0)),
            pl.BlockSpec((128, 65), lambda g: (0, 0)),
            pl.BlockSpec((1, 65), lambda g: (0, 0)),
            pl.BlockSpec((65, 2), lambda g: (0, 0)),
            pl.BlockSpec((1, 2), lambda g: (0, 0)),
        ],
        out_specs=pl.BlockSpec((BB, 2), lambda g: (g, 0)),
        out_shape=jax.ShapeDtypeStruct((B, 2), jnp.float32),
    )(H, mb1, M2, mb2, M3, mb3, M4, mb4)


def kernel(x, edge_index, edge_type, idx, attt, Wrel1, Wroot1, b1, Wrel2,
           Wroot2, b2, Wg1, bg1, Wg2, bg2, M1, mb1, M2, mb2, M3, mb3, M4,
           mb4, features3):
    src, dst = edge_index[0], edge_index[1]
    et = edge_type
    epad = E_ROWS_PAD * CH - E
    zpad = jnp.zeros((epad,), jnp.int32)
    tpad = jnp.full((epad,), N, jnp.int32)
    gidx = jnp.concatenate([et * N + src, zpad]).reshape(-1, CH)
    sidx = jnp.concatenate([et * AR + dst, tpad]).reshape(-1, CH)
    src_p = jnp.concatenate([src, zpad]).reshape(-1, CH)
    dst_p = jnp.concatenate([dst, tpad]).reshape(-1, CH)
    bpad = P_IDX_PAD - B
    aa_p = jnp.concatenate([idx[0], jnp.zeros((bpad,), jnp.int32)])
    bb_p = jnp.concatenate([idx[1], jnp.zeros((bpad,), jnp.int32)])

    a0, a1 = attt[0], attt[1]
    Ma = jnp.concatenate([a0 * M1[0:64], a1 * M1[64:96], a0 * M1[96:160],
                          a1 * M1[160:192], M1[192:704]], 0)
    Mb = jnp.concatenate([a0 * M1[704:768], a1 * M1[768:800], a0 * M1[800:864],
                          a1 * M1[864:896], M1[896:1408]], 0)
    Mab = jnp.stack([Ma, Mb])

    b1r = b1.reshape(1, 64)
    bg1r = bg1.reshape(1, 64)
    b2r = b2.reshape(1, 32)
    bg2r = bg2.reshape(1, 32)
    mb1r = mb1.reshape(1, 256)
    mb2r = mb2.reshape(1, 128)
    mb3r = mb3.reshape(1, 65)
    mb4r = mb4.reshape(1, 2)

    z16 = jnp.zeros((2 * AR // NS, 16), jnp.float32)
    z64a = jnp.zeros((2 * AR // NS, 64), jnp.float32)
    z64 = jnp.zeros((A2 // NS, 64), jnp.float32)
    z32a = jnp.zeros((2 * AR // NS, 32), jnp.float32)
    z32b = jnp.zeros((A2 // NS, 32), jnp.float32)
    ones16 = jnp.ones((CH, 16), jnp.float32)

    CNT = _sc_count(sidx, ones16, z16).reshape(NC, 2, AR, 16)
    T1 = _k1(x, Wrel1)
    S1 = _sc_seg1(T1.reshape(2 * N, 64), gidx, sidx, z64a).reshape(NC, 2, AR, 64)
    x1_o, hg1, T2, aux = _k2(x, S1, CNT, Wroot1, b1r, Wg1, Wrel2)
    G1 = _sc_seg2(hg1, src_p, dst_p, z64)
    S2 = _sc_seg3(T2.reshape(2 * N, 32), gidx, sidx, z32a).reshape(NC, 2, AR, 32)
    x1_o1, hg2 = _k3(G1, hg1, aux, bg1r, Wg2)
    G2 = _sc_seg4(hg2, src_p, dst_p, z32b)
    x2_o, Pa, Pb = _k4(S2, x1_o, Wroot2, b2r, aux, G2, hg2, bg2r, x1_o1,
                       features3, Mab)
    H = _sc_pair(Pa, Pb, aa_p, bb_p)
    logp = _k5(H, mb1r, M2, mb2r, M3, mb3r, M4, mb4r)
    return (logp[:B], x2_o)


# speedup vs baseline: 9.8014x; 1.0050x over previous
import functools

import jax
import jax.numpy as jnp
from jax import lax
from jax.experimental import pallas as pl
from jax.experimental.pallas import tpu as pltpu
from jax.experimental.pallas import tpu_sc as plsc

N = 10000
E = 320000
B = 100000

NC, NS = 2, 16
NW = NC * NS
CH = 128

CH5 = 64

E_PAD = ((E + 2 * NW * CH - 1) // (2 * NW * CH)) * (2 * NW * CH)
B_PAD = ((B + 2 * NW * CH5 - 1) // (2 * NW * CH5)) * (2 * NW * CH5)

ECW = 160
E_ROWS_PAD = 2680
PC0 = 76
PC1 = 24
P_IDX_PAD = (17 * PC0 + 15 * PC1) * CH5
AR = N + 48
A2 = N + 112

BN = 1000
GRID_N = N // BN


def _make_sc_segsum(table_rows, width, acc_rows, ec0):
    ec1 = ECW - ec0
    stripe = acc_rows // NS
    mesh = plsc.VectorSubcoreMesh(core_axis_name="c", subcore_axis_name="s")

    @functools.partial(
        pl.kernel, mesh=mesh,
        compiler_params=pltpu.CompilerParams(use_tc_tiling_on_sc=False),
        out_type=jax.ShapeDtypeStruct((NC, acc_rows, width), jnp.float32),
        scratch_types=[
            pltpu.VMEM((ec0, CH), jnp.int32),
            pltpu.VMEM((ec0, CH), jnp.int32),
            pltpu.VMEM((CH, width), jnp.float32),
            pltpu.VMEM((CH, width), jnp.float32),
            pltpu.VMEM_SHARED((acc_rows, width), jnp.float32),
            pltpu.SemaphoreType.DMA,
            pltpu.SemaphoreType.DMA,
        ],
    )
    def k(table, gidx, sidx, zeros, out, gv, sv, r0, r1, acc, s0, s1):
        c = lax.axis_index("c")
        s = lax.axis_index("s")
        pltpu.sync_copy(zeros, acc.at[pl.ds(s * stripe, stripe)])
        n_ch = jnp.where(c == 0, ec0, ec1)
        rb = jnp.where(c == 0, s * ec0, NS * ec0 + s * ec1)
        h1 = ec0 // 2
        h2 = ec0 - h1
        pltpu.sync_copy(gidx.at[pl.ds(rb, h1)], gv.at[pl.ds(0, h1)])
        pltpu.sync_copy(gidx.at[pl.ds(rb + h1, h2)], gv.at[pl.ds(h1, h2)])
        pltpu.sync_copy(sidx.at[pl.ds(rb, h1)], sv.at[pl.ds(0, h1)])
        pltpu.sync_copy(sidx.at[pl.ds(rb + h1, h2)], sv.at[pl.ds(h1, h2)])
        plsc.subcore_barrier()

        def fire(chunk, buf, sem):
            pltpu.async_copy(table.at[gv.at[chunk]], buf, sem)

        def wait(buf, sem):
            pltpu.make_async_copy(table.at[pl.ds(0, CH)], buf, sem).wait()

        def scat(buf, chunk):
            pltpu.sync_copy(buf, acc.at[sv.at[chunk]], add=True)

        fire(0, r0, s0)
        fire(1, r1, s1)

        def body(j2, carry):
            j = j2 * 2
            wait(r0, s0)
            scat(r0, j)
            fire(jnp.minimum(j + 2, n_ch - 1), r0, s0)
            wait(r1, s1)
            scat(r1, j + 1)
            fire(jnp.minimum(j + 3, n_ch - 1), r1, s1)
            return carry

        lax.fori_loop(0, n_ch // 2, body, 0)
        wait(r0, s0)
        wait(r1, s1)
        plsc.subcore_barrier()
        pltpu.sync_copy(acc.at[pl.ds(s * stripe, stripe)],
                        out.at[c, pl.ds(s * stripe, stripe)])

    return k


def _make_sc_pair_gather(width):
    kk = width // 16
    mesh = plsc.VectorSubcoreMesh(core_axis_name="c", subcore_axis_name="s")

    @functools.partial(
        pl.kernel, mesh=mesh,
        compiler_params=pltpu.CompilerParams(use_tc_tiling_on_sc=True),
        out_type=jax.ShapeDtypeStruct((B_PAD, width), jnp.float32),
        scratch_types=[
            pltpu.VMEM((PC0 * CH5,), jnp.int32),
            pltpu.VMEM((PC0 * CH5,), jnp.int32),
            pltpu.VMEM((CH5, width), jnp.float32),
            pltpu.VMEM((CH5, width), jnp.float32),
            pltpu.VMEM((CH5, width), jnp.float32),
            pltpu.VMEM((CH5, width), jnp.float32),
            pltpu.SemaphoreType.DMA,
            pltpu.SemaphoreType.DMA,
            pltpu.SemaphoreType.DMA,
            pltpu.SemaphoreType.DMA,
        ],
    )
    def k(pa, pb, aidx, bidx, out, av, bv, ra0, rb0, ra1, rb1,
          sa0, sb0, sa1, sb1):
        c = lax.axis_index("c")
        s = lax.axis_index("s")
        n_ch = jnp.where(c == 0, PC0, PC1)
        cb = jnp.where(c == 0, s * PC0, NS * PC0 + s * PC1)
        base = cb * CH5
        pltpu.sync_copy(aidx.at[pl.ds(base, PC0 * CH5)], av)
        pltpu.sync_copy(bidx.at[pl.ds(base, PC0 * CH5)], bv)

        def fire(chunk, ra, rb, sa, sb):
            pltpu.async_copy(pa.at[av.at[pl.ds(chunk * CH5, CH5)]], ra, sa)
            pltpu.async_copy(pb.at[bv.at[pl.ds(chunk * CH5, CH5)]], rb, sb)

        def wait(ra, rb, sa, sb):
            pltpu.make_async_copy(pa.at[pl.ds(0, CH5)], ra, sa).wait()
            pltpu.make_async_copy(pb.at[pl.ds(0, CH5)], rb, sb).wait()

        def process(ra, rb, chunk):
            def add_row(r, carry2):
                for q in range(kk):
                    ra[r, pl.ds(q * 16, 16)] = (
                        ra[r, pl.ds(q * 16, 16)] + rb[r, pl.ds(q * 16, 16)])
                return carry2

            lax.fori_loop(0, CH5, add_row, 0)
            pltpu.sync_copy(ra, out.at[pl.ds(base + chunk * CH5, CH5)])

        fire(0, ra0, rb0, sa0, sb0)
        fire(1, ra1, rb1, sa1, sb1)

        def body(j2, carry):
            j = j2 * 2
            wait(ra0, rb0, sa0, sb0)
            process(ra0, rb0, j)
            fire(jnp.minimum(j + 2, n_ch - 1), ra0, rb0, sa0, sb0)
            wait(ra1, rb1, sa1, sb1)
            process(ra1, rb1, j + 1)
            fire(jnp.minimum(j + 3, n_ch - 1), ra1, rb1, sa1, sb1)
            return carry

        lax.fori_loop(0, n_ch // 2, body, 0)
        wait(ra0, rb0, sa0, sb0)
        wait(ra1, rb1, sa1, sb1)

    return k


def _make_sc_count():
    ec0, ec1 = 104, 56
    acc_rows = 2 * AR
    stripe = acc_rows // NS
    mesh = plsc.VectorSubcoreMesh(core_axis_name="c", subcore_axis_name="s")

    @functools.partial(
        pl.kernel, mesh=mesh,
        compiler_params=pltpu.CompilerParams(use_tc_tiling_on_sc=False),
        out_type=jax.ShapeDtypeStruct((NC, acc_rows, 16), jnp.float32),
        scratch_types=[
            pltpu.VMEM((ec0, CH), jnp.int32),
            pltpu.VMEM((CH, 16), jnp.float32),
            pltpu.VMEM_SHARED((acc_rows, 16), jnp.float32),
        ],
    )
    def k(sidx, ones, zeros, out, sv, ov, acc):
        c = lax.axis_index("c")
        s = lax.axis_index("s")
        pltpu.sync_copy(zeros, acc.at[pl.ds(s * stripe, stripe)])
        pltpu.sync_copy(ones, ov)
        n_ch = jnp.where(c == 0, ec0, ec1)
        rb = jnp.where(c == 0, s * ec0, NS * ec0 + s * ec1)
        h1 = ec0 // 2
        h2 = ec0 - h1
        pltpu.sync_copy(sidx.at[pl.ds(rb, h1)], sv.at[pl.ds(0, h1)])
        pltpu.sync_copy(sidx.at[pl.ds(rb + h1, h2)], sv.at[pl.ds(h1, h2)])
        plsc.subcore_barrier()

        def body(j, carry):
            pltpu.sync_copy(ov, acc.at[sv.at[j]], add=True)
            return carry

        lax.fori_loop(0, n_ch, body, 0)
        plsc.subcore_barrier()
        pltpu.sync_copy(acc.at[pl.ds(s * stripe, stripe)],
                        out.at[c, pl.ds(s * stripe, stripe)])

    return k


_sc_count = _make_sc_count()
_sc_seg1 = _make_sc_segsum(2 * N, 64, 2 * AR, 128)
_sc_seg2 = _make_sc_segsum(N, 64, A2, 136)
_sc_seg3 = _make_sc_segsum(2 * N, 32, 2 * AR, 128)
_sc_seg4 = _make_sc_segsum(N, 32, A2, 120)
_sc_pair = _make_sc_pair_gather(256)


def _k1_body(x_ref, w_ref, t_ref):
    xb = x_ref[...]
    t_ref[0] = jnp.dot(xb, w_ref[0], preferred_element_type=jnp.float32)
    t_ref[1] = jnp.dot(xb, w_ref[1], preferred_element_type=jnp.float32)


def _k1(x, Wrel1):
    return pl.pallas_call(
        _k1_body,
        grid=(GRID_N,),
        in_specs=[
            pl.BlockSpec((BN, 128), lambda g: (g, 0)),
            pl.BlockSpec((2, 128, 64), lambda g: (0, 0, 0)),
        ],
        out_specs=pl.BlockSpec((2, BN, 64), lambda g: (0, g, 0)),
        out_shape=jax.ShapeDtypeStruct((2, N, 64), jnp.float32),
    )(x, Wrel1)


def _k2_body(x_ref, s1_ref, cnt_ref, wr_ref, b1_ref, wg_ref, w2_ref,
             x1_ref, hg1_ref, t2_ref, aux_ref):
    S0 = s1_ref[0, 0] + s1_ref[1, 0]
    S1 = s1_ref[0, 1] + s1_ref[1, 1]
    c0 = cnt_ref[0, 0, :, 0] + cnt_ref[1, 0, :, 0]
    c1 = cnt_ref[0, 1, :, 0] + cnt_ref[1, 1, :, 0]
    dinv = lax.rsqrt(c0 + c1 + 1.0)
    ic0 = 1.0 / jnp.maximum(c0, 1.0)
    ic1 = 1.0 / jnp.maximum(c1, 1.0)
    xb = x_ref[...]
    x1 = jnp.dot(xb, wr_ref[...], preferred_element_type=jnp.float32) + b1_ref[...]
    x1 = x1 + S0 * ic0[:, None] + S1 * ic1[:, None]
    x1 = jnp.maximum(x1, 0.0)
    x1_ref[...] = x1
    hg1_ref[...] = dinv[:, None] * jnp.dot(xb, wg_ref[...],
                                           preferred_element_type=jnp.float32)
    t2_ref[0] = jnp.dot(x1, w2_ref[0], preferred_element_type=jnp.float32)
    t2_ref[1] = jnp.dot(x1, w2_ref[1], preferred_element_type=jnp.float32)
    aux_ref[...] = jnp.concatenate(
        [dinv[:, None], ic0[:, None], ic1[:, None],
         jnp.zeros((BN, 5), jnp.float32)], -1)


def _k2(x, S1, CNT, Wroot1, b1, Wg1, Wrel2):
    return pl.pallas_call(
        _k2_body,
        grid=(GRID_N,),
        in_specs=[
            pl.BlockSpec((BN, 128), lambda g: (g, 0)),
            pl.BlockSpec((2, 2, BN, 64), lambda g: (0, 0, g, 0)),
            pl.BlockSpec((2, 2, BN, 16), lambda g: (0, 0, g, 0)),
            pl.BlockSpec((128, 64), lambda g: (0, 0)),
            pl.BlockSpec((1, 64), lambda g: (0, 0)),
            pl.BlockSpec((128, 64), lambda g: (0, 0)),
            pl.BlockSpec((2, 64, 32), lambda g: (0, 0, 0)),
        ],
        out_specs=[
            pl.BlockSpec((BN, 64), lambda g: (g, 0)),
            pl.BlockSpec((BN, 64), lambda g: (g, 0)),
            pl.BlockSpec((2, BN, 32), lambda g: (0, g, 0)),
            pl.BlockSpec((BN, 8), lambda g: (g, 0)),
        ],
        out_shape=[
            jax.ShapeDtypeStruct((N, 64), jnp.float32),
            jax.ShapeDtypeStruct((N, 64), jnp.float32),
            jax.ShapeDtypeStruct((2, N, 32), jnp.float32),
            jax.ShapeDtypeStruct((N, 8), jnp.float32),
        ],
    )(x, S1, CNT, Wroot1, b1, Wg1, Wrel2)


def _k3_body(g1_ref, hg1_ref, aux_ref, bg1_ref, wg2_ref, x11_ref, hg2_ref):
    dinv = aux_ref[...][:, 0]
    G = g1_ref[0] + g1_ref[1]
    x11 = jnp.maximum(dinv[:, None] * (G + hg1_ref[...]) + bg1_ref[...], 0.0)
    x11_ref[...] = x11
    hg2_ref[...] = dinv[:, None] * jnp.dot(x11, wg2_ref[...],
                                           preferred_element_type=jnp.float32)


def _k3(G1, hg1, aux, bg1, Wg2):
    return pl.pallas_call(
        _k3_body,
        grid=(GRID_N,),
        in_specs=[
            pl.BlockSpec((2, BN, 64), lambda g: (0, g, 0)),
            pl.BlockSpec((BN, 64), lambda g: (g, 0)),
            pl.BlockSpec((BN, 8), lambda g: (g, 0)),
            pl.BlockSpec((1, 64), lambda g: (0, 0)),
            pl.BlockSpec((64, 32), lambda g: (0, 0)),
        ],
        out_specs=[
            pl.BlockSpec((BN, 64), lambda g: (g, 0)),
            pl.BlockSpec((BN, 32), lambda g: (g, 0)),
        ],
        out_shape=[
            jax.ShapeDtypeStruct((N, 64), jnp.float32),
            jax.ShapeDtypeStruct((N, 32), jnp.float32),
        ],
    )(G1, hg1, aux, bg1, Wg2)


def _k4_body(s2_ref, x1_ref, wr2_ref, b2_ref, aux_ref, g2_ref, hg2_ref,
             bg2_ref, x11_ref, f3_ref, mab_ref, x2_ref, pa_ref, pb_ref):
    aux = aux_ref[...]
    dinv = aux[:, 0]
    ic0 = aux[:, 1]
    ic1 = aux[:, 2]
    S0 = s2_ref[0, 0] + s2_ref[1, 0]
    S1 = s2_ref[0, 1] + s2_ref[1, 1]
    x1 = x1_ref[...]
    x2 = jnp.dot(x1, wr2_ref[...], preferred_element_type=jnp.float32) + b2_ref[...]
    x2 = x2 + S0 * ic0[:, None] + S1 * ic1[:, None]
    x2_ref[...] = x2
    G2 = g2_ref[0] + g2_ref[1]
    x21 = dinv[:, None] * (G2 + hg2_ref[...]) + bg2_ref[...]
    Z = jnp.concatenate([x11_ref[...], x21, x1, x2, f3_ref[...]], -1)
    pa_ref[...] = jnp.dot(Z, mab_ref[0], preferred_element_type=jnp.float32)
    pb_ref[...] = jnp.dot(Z, mab_ref[1], preferred_element_type=jnp.float32)


def _k4(S2, x1_o, Wroot2, b2, aux, G2, hg2, bg2, x1_o1, f3, Mab):
    return pl.pallas_call(
        _k4_body,
        grid=(GRID_N,),
        in_specs=[
            pl.BlockSpec((2, 2, BN, 32), lambda g: (0, 0, g, 0)),
            pl.BlockSpec((BN, 64), lambda g: (g, 0)),
            pl.BlockSpec((64, 32), lambda g: (0, 0)),
            pl.BlockSpec((1, 32), lambda g: (0, 0)),
            pl.BlockSpec((BN, 8), lambda g: (g, 0)),
            pl.BlockSpec((2, BN, 32), lambda g: (0, g, 0)),
            pl.BlockSpec((BN, 32), lambda g: (g, 0)),
            pl.BlockSpec((1, 32), lambda g: (0, 0)),
            pl.BlockSpec((BN, 64), lambda g: (g, 0)),
            pl.BlockSpec((BN, 512), lambda g: (g, 0)),
            pl.BlockSpec((2, 704, 256), lambda g: (0, 0, 0)),
        ],
        out_specs=[
            pl.BlockSpec((BN, 32), lambda g: (g, 0)),
            pl.BlockSpec((BN, 256), lambda g: (g, 0)),
            pl.BlockSpec((BN, 256), lambda g: (g, 0)),
        ],
        out_shape=[
            jax.ShapeDtypeStruct((N, 32), jnp.float32),
            jax.ShapeDtypeStruct((N, 256), jnp.float32),
            jax.ShapeDtypeStruct((N, 256), jnp.float32),
        ],
    )(S2, x1_o, Wroot2, b2, aux, G2, hg2, bg2, x1_o1, f3, Mab)


BB = 1000
GRID_B = B // BB


def _k5_body(h_ref, mb1_ref, m2_ref, mb2_ref, m3_ref, mb3_ref, m4_ref,
             mb4_ref, log_ref):
    h = h_ref[...] + mb1_ref[...]
    e1 = jnp.where(h > 0, h, jnp.exp(h) - 1.0)
    t = jnp.dot(e1, m2_ref[...], preferred_element_type=jnp.float32) + mb2_ref[...]
    e2 = jnp.where(t > 0, t, jnp.exp(t) - 1.0)
    m34 = jnp.dot(m3_ref[...], m4_ref[...], preferred_element_type=jnp.float32)
    mb34 = jnp.dot(mb3_ref[...], m4_ref[...],
                   preferred_element_type=jnp.float32) + mb4_ref[...]
    log_ref[...] = jnp.dot(e2, m34, preferred_element_type=jnp.float32) + mb34


def _k5(H, mb1, M2, mb2, M3, mb3, M4, mb4):
    return pl.pallas_call(
        _k5_body,
        grid=(GRID_B,),
        in_specs=[
            pl.BlockSpec((BB, 256), lambda g: (g, 0)),
            pl.BlockSpec((1, 256), lambda g: (0, 0)),
            pl.BlockSpec((256, 128), lambda g: (0, 0)),
            pl.BlockSpec((1, 128), lambda g: (0, 0)),
            pl.BlockSpec((128, 65), lambda g: (0, 0)),
            pl.BlockSpec((1, 65), lambda g: (0, 0)),
            pl.BlockSpec((65, 2), lambda g: (0, 0)),
            pl.BlockSpec((1, 2), lambda g: (0, 0)),
        ],
        out_specs=pl.BlockSpec((BB, 2), lambda g: (g, 0)),
        out_shape=jax.ShapeDtypeStruct((B, 2), jnp.float32),
    )(H, mb1, M2, mb2, M3, mb3, M4, mb4)


def kernel(x, edge_index, edge_type, idx, attt, Wrel1, Wroot1, b1, Wrel2,
           Wroot2, b2, Wg1, bg1, Wg2, bg2, M1, mb1, M2, mb2, M3, mb3, M4,
           mb4, features3):
    src, dst = edge_index[0], edge_index[1]
    et = edge_type
    epad = E_ROWS_PAD * CH - E
    zpad = jnp.zeros((epad,), jnp.int32)
    tpad = jnp.full((epad,), N, jnp.int32)
    gidx = jnp.concatenate([et * N + src, zpad]).reshape(-1, CH)
    sidx = jnp.concatenate([et * AR + dst, tpad]).reshape(-1, CH)
    src_p = jnp.concatenate([src, zpad]).reshape(-1, CH)
    dst_p = jnp.concatenate([dst, tpad]).reshape(-1, CH)
    bpad = P_IDX_PAD - B
    aa_p = jnp.concatenate([idx[0], jnp.zeros((bpad,), jnp.int32)])
    bb_p = jnp.concatenate([idx[1], jnp.zeros((bpad,), jnp.int32)])

    a0, a1 = attt[0], attt[1]
    Ma = jnp.concatenate([a0 * M1[0:64], a1 * M1[64:96], a0 * M1[96:160],
                          a1 * M1[160:192], M1[192:704]], 0)
    Mb = jnp.concatenate([a0 * M1[704:768], a1 * M1[768:800], a0 * M1[800:864],
                          a1 * M1[864:896], M1[896:1408]], 0)
    Mab = jnp.stack([Ma, Mb])

    b1r = b1.reshape(1, 64)
    bg1r = bg1.reshape(1, 64)
    b2r = b2.reshape(1, 32)
    bg2r = bg2.reshape(1, 32)
    mb1r = mb1.reshape(1, 256)
    mb2r = mb2.reshape(1, 128)
    mb3r = mb3.reshape(1, 65)
    mb4r = mb4.reshape(1, 2)

    z16 = jnp.zeros((2 * AR // NS, 16), jnp.float32)
    z64a = jnp.zeros((2 * AR // NS, 64), jnp.float32)
    z64 = jnp.zeros((A2 // NS, 64), jnp.float32)
    z32a = jnp.zeros((2 * AR // NS, 32), jnp.float32)
    z32b = jnp.zeros((A2 // NS, 32), jnp.float32)
    ones16 = jnp.ones((CH, 16), jnp.float32)

    CNT = _sc_count(sidx, ones16, z16).reshape(NC, 2, AR, 16)
    T1 = _k1(x, Wrel1)
    S1 = _sc_seg1(T1.reshape(2 * N, 64), gidx, sidx, z64a).reshape(NC, 2, AR, 64)
    x1_o, hg1, T2, aux = _k2(x, S1, CNT, Wroot1, b1r, Wg1, Wrel2)
    G1 = _sc_seg2(hg1, src_p, dst_p, z64)
    S2 = _sc_seg3(T2.reshape(2 * N, 32), gidx, sidx, z32a).reshape(NC, 2, AR, 32)
    x1_o1, hg2 = _k3(G1, hg1, aux, bg1r, Wg2)
    G2 = _sc_seg4(hg2, src_p, dst_p, z32b)
    x2_o, Pa, Pb = _k4(S2, x1_o, Wroot2, b2r, aux, G2, hg2, bg2r, x1_o1,
                       features3, Mab)
    H = _sc_pair(Pa, Pb, aa_p, bb_p)
    logp = _k5(H, mb1r, M2, mb2r, M3, mb3r, M4, mb4r)
    return (logp[:B], x2_o)
